# Initial kernel scaffold; baseline (speedup 1.0000x reference)
import jax
import jax.numpy as jnp
from jax.experimental import pallas as pl


def kernel(x_t, h_lstm, c_lstm, slots, ptr, mem_value_w, mem_value_b, mem_det_w, mem_det_b, mem_pos_emb, mem_weights, mem_proj_w, mem_proj_b, W_ih_w, W_ih_b, W_hh_w):
    raise NotImplementedError("write your pallas kernel here")


# speedup vs baseline: 3.0466x; 3.0466x over previous
import jax
import jax.numpy as jnp
from jax.experimental import pallas as pl
from jax.experimental.pallas import tpu as pltpu

B = 4096
D = 128
H = 128
S = 200
TAU = 0.5
K = 32
RB = K * S


def _cell_kernel(x_ref, h_ref, c_ref, slots_ref, ptr_ref, mw_tiled_ref,
                 value_wt_ref, value_b_ref, det_wt_ref, det_b_ref,
                 pos_emb_ref, proj_wt_ref, proj_b_ref,
                 wih_x_t_ref, wih_h_t_ref, wih_b_ref, whh_t_ref,
                 h_new_ref, c_new_ref, slots_new_ref, ptr_new_ref,
                 kw_ref, pos_c_ref):
    @pl.when(pl.program_id(0) == 0)
    def _init():
        mw = mw_tiled_ref[...]
        ex = jnp.exp(mw - jnp.max(mw, axis=1, keepdims=True))
        z = jnp.sum(ex, axis=1, keepdims=True) * (1.0 / K)
        wt = ex / z
        r_iota = jax.lax.broadcasted_iota(jnp.int32, (K, RB), 1)
        j_iota = jax.lax.broadcasted_iota(jnp.int32, (K, RB), 0)
        seg = (r_iota // S) == j_iota
        kw_ref[...] = seg.astype(jnp.float32) * wt
        pos_c_ref[...] = jnp.dot(wt[:, :S], pos_emb_ref[...],
                                 preferred_element_type=jnp.float32)

    x = x_ref[...]
    det = jnp.dot(x, det_wt_ref[...]) + det_b_ref[...]
    e_t = jax.nn.sigmoid(det)
    v = jnp.dot(x, value_wt_ref[...],
                preferred_element_type=jnp.float32) + value_b_ref[...]

    slots_new_ref[...] = slots_ref[...]
    for j in range(K):
        pj = ptr_ref[0, 0, j]
        cond = e_t[j, 0] > TAU

        @pl.when(cond)
        def _store(j=j, pj=pj):
            slots_new_ref[pl.ds(j * S + pj, 1), :] = v[j:j + 1, :]

        ptr_new_ref[0, 0, j] = jnp.where(cond, (pj + 1) % S, pj)

    fused = jnp.dot(kw_ref[...], slots_new_ref[...],
                    preferred_element_type=jnp.float32) + pos_c_ref[...]
    h_mem = jnp.dot(fused, proj_wt_ref[...],
                    preferred_element_type=jnp.float32) + proj_b_ref[...]
    gates = (jnp.dot(x, wih_x_t_ref[...], preferred_element_type=jnp.float32)
             + jnp.dot(h_mem, wih_h_t_ref[...], preferred_element_type=jnp.float32)
             + jnp.dot(h_ref[...], whh_t_ref[...], preferred_element_type=jnp.float32)
             + wih_b_ref[...])
    i_g = gates[:, 0 * H:1 * H]
    f_g = gates[:, 1 * H:2 * H]
    g_g = gates[:, 2 * H:3 * H]
    o_g = gates[:, 3 * H:4 * H]
    c_new = jax.nn.sigmoid(f_g) * c_ref[...] + jax.nn.sigmoid(i_g) * jnp.tanh(g_g)
    h_new_ref[...] = jax.nn.sigmoid(o_g) * jnp.tanh(c_new)
    c_new_ref[...] = c_new


def kernel(x_t, h_lstm, c_lstm, slots, ptr, mem_value_w, mem_value_b,
           mem_det_w, mem_det_b, mem_pos_emb, mem_weights, mem_proj_w,
           mem_proj_b, W_ih_w, W_ih_b, W_hh_w):
    nb = B // K
    slots2d = slots.reshape(B * S, D)
    mw_tiled = jnp.tile(mem_weights.reshape(1, S), (1, K))
    row_spec = lambda cols: pl.BlockSpec((K, cols), lambda i: (i, 0))
    full_spec = lambda r, c: pl.BlockSpec((r, c), lambda i: (0, 0))
    out = pl.pallas_call(
        _cell_kernel,
        grid=(nb,),
        in_specs=[
            row_spec(D),
            row_spec(H),
            row_spec(H),
            pl.BlockSpec((RB, D), lambda i: (i, 0)),
            pl.BlockSpec((1, 1, K), lambda i: (i, 0, 0),
                         memory_space=pltpu.SMEM),
            full_spec(1, RB),
            full_spec(D, D),
            full_spec(1, D),
            full_spec(D, 1),
            full_spec(1, 1),
            full_spec(S, D),
            full_spec(D, H),
            full_spec(1, H),
            full_spec(D, 4 * H),
            full_spec(H, 4 * H),
            full_spec(1, 4 * H),
            full_spec(H, 4 * H),
        ],
        out_specs=[
            row_spec(H),
            row_spec(H),
            pl.BlockSpec((RB, D), lambda i: (i, 0)),
            pl.BlockSpec((1, 1, K), lambda i: (i, 0, 0),
                         memory_space=pltpu.SMEM),
        ],
        out_shape=[
            jax.ShapeDtypeStruct((B, H), jnp.float32),
            jax.ShapeDtypeStruct((B, H), jnp.float32),
            jax.ShapeDtypeStruct((B * S, D), jnp.float32),
            jax.ShapeDtypeStruct((nb, 1, K), jnp.int32),
        ],
        scratch_shapes=[
            pltpu.VMEM((K, RB), jnp.float32),
            pltpu.VMEM((1, D), jnp.float32),
        ],
    )(
        x_t, h_lstm, c_lstm, slots2d, ptr.reshape(nb, 1, K), mw_tiled,
        mem_value_w.T, mem_value_b.reshape(1, D),
        mem_det_w.T, mem_det_b.reshape(1, 1),
        mem_pos_emb,
        mem_proj_w.T, mem_proj_b.reshape(1, H),
        W_ih_w[:, :D].T, W_ih_w[:, D:].T, W_ih_b.reshape(1, 4 * H),
        W_hh_w.T,
    )
    h_new, c_new, slots_new2d, ptr_new = out
    return (h_new, c_new, slots_new2d.reshape(B, S, D), ptr_new.reshape(B))


# speedup vs baseline: 3.5270x; 1.1577x over previous
import jax
import jax.numpy as jnp
from jax.experimental import pallas as pl
from jax.experimental.pallas import tpu as pltpu

B = 4096
D = 128
H = 128
S = 200
TAU = 0.5
K = 64
RB = K * S


def _cell_kernel(x_ref, h_ref, c_ref, slots_ref, ptr_ref, mw_tiled_ref,
                 value_wt_ref, value_b_ref, det_wt_ref, det_b_ref,
                 pos_emb_ref, proj_wt_ref, proj_b_ref,
                 wih_x_t_ref, wih_h_t_ref, wih_b_ref, whh_t_ref,
                 h_new_ref, c_new_ref, slots_new_ref, ptr_new_ref,
                 kw_ref, pos_c_ref):
    @pl.when(pl.program_id(0) == 0)
    def _init():
        mw = mw_tiled_ref[...]
        ex = jnp.exp(mw - jnp.max(mw, axis=1, keepdims=True))
        z = jnp.sum(ex, axis=1, keepdims=True) * (1.0 / K)
        wt = ex / z
        r_iota = jax.lax.broadcasted_iota(jnp.int32, (K, RB), 1)
        j_iota = jax.lax.broadcasted_iota(jnp.int32, (K, RB), 0)
        seg = (r_iota // S) == j_iota
        kw_ref[...] = seg.astype(jnp.float32) * wt
        pos_c_ref[...] = jnp.dot(wt[:, :S], pos_emb_ref[...],
                                 preferred_element_type=jnp.float32)

    x = x_ref[...]
    det = jnp.dot(x, det_wt_ref[...]) + det_b_ref[...]
    e_t = jax.nn.sigmoid(det)
    v = jnp.dot(x, value_wt_ref[...],
                preferred_element_type=jnp.float32) + value_b_ref[...]

    slots_new_ref[...] = slots_ref[...]
    for j in range(K):
        pj = ptr_ref[0, 0, j]
        cond = e_t[j, 0] > TAU

        @pl.when(cond)
        def _store(j=j, pj=pj):
            slots_new_ref[pl.ds(j * S + pj, 1), :] = v[j:j + 1, :]

        ptr_new_ref[0, 0, j] = jnp.where(cond, (pj + 1) % S, pj)

    fused = jnp.dot(kw_ref[...], slots_new_ref[...],
                    preferred_element_type=jnp.float32) + pos_c_ref[...]
    h_mem = jnp.dot(fused, proj_wt_ref[...],
                    preferred_element_type=jnp.float32) + proj_b_ref[...]
    gates = (jnp.dot(x, wih_x_t_ref[...], preferred_element_type=jnp.float32)
             + jnp.dot(h_mem, wih_h_t_ref[...], preferred_element_type=jnp.float32)
             + jnp.dot(h_ref[...], whh_t_ref[...], preferred_element_type=jnp.float32)
             + wih_b_ref[...])
    i_g = gates[:, 0 * H:1 * H]
    f_g = gates[:, 1 * H:2 * H]
    g_g = gates[:, 2 * H:3 * H]
    o_g = gates[:, 3 * H:4 * H]
    c_new = jax.nn.sigmoid(f_g) * c_ref[...] + jax.nn.sigmoid(i_g) * jnp.tanh(g_g)
    h_new_ref[...] = jax.nn.sigmoid(o_g) * jnp.tanh(c_new)
    c_new_ref[...] = c_new


def kernel(x_t, h_lstm, c_lstm, slots, ptr, mem_value_w, mem_value_b,
           mem_det_w, mem_det_b, mem_pos_emb, mem_weights, mem_proj_w,
           mem_proj_b, W_ih_w, W_ih_b, W_hh_w):
    nb = B // K
    slots2d = slots.reshape(B * S, D)
    mw_tiled = jnp.tile(mem_weights.reshape(1, S), (1, K))
    row_spec = lambda cols: pl.BlockSpec((K, cols), lambda i: (i, 0))
    full_spec = lambda r, c: pl.BlockSpec((r, c), lambda i: (0, 0))
    out = pl.pallas_call(
        _cell_kernel,
        grid=(nb,),
        in_specs=[
            row_spec(D),
            row_spec(H),
            row_spec(H),
            pl.BlockSpec((RB, D), lambda i: (i, 0)),
            pl.BlockSpec((1, 1, K), lambda i: (i, 0, 0),
                         memory_space=pltpu.SMEM),
            full_spec(1, RB),
            full_spec(D, D),
            full_spec(1, D),
            full_spec(D, 1),
            full_spec(1, 1),
            full_spec(S, D),
            full_spec(D, H),
            full_spec(1, H),
            full_spec(D, 4 * H),
            full_spec(H, 4 * H),
            full_spec(1, 4 * H),
            full_spec(H, 4 * H),
        ],
        out_specs=[
            row_spec(H),
            row_spec(H),
            pl.BlockSpec((RB, D), lambda i: (i, 0)),
            pl.BlockSpec((1, 1, K), lambda i: (i, 0, 0),
                         memory_space=pltpu.SMEM),
        ],
        out_shape=[
            jax.ShapeDtypeStruct((B, H), jnp.float32),
            jax.ShapeDtypeStruct((B, H), jnp.float32),
            jax.ShapeDtypeStruct((B * S, D), jnp.float32),
            jax.ShapeDtypeStruct((nb, 1, K), jnp.int32),
        ],
        scratch_shapes=[
            pltpu.VMEM((K, RB), jnp.float32),
            pltpu.VMEM((1, D), jnp.float32),
        ],
    )(
        x_t, h_lstm, c_lstm, slots2d, ptr.reshape(nb, 1, K), mw_tiled,
        mem_value_w.T, mem_value_b.reshape(1, D),
        mem_det_w.T, mem_det_b.reshape(1, 1),
        mem_pos_emb,
        mem_proj_w.T, mem_proj_b.reshape(1, H),
        W_ih_w[:, :D].T, W_ih_w[:, D:].T, W_ih_b.reshape(1, 4 * H),
        W_hh_w.T,
    )
    h_new, c_new, slots_new2d, ptr_new = out
    return (h_new, c_new, slots_new2d.reshape(B, S, D), ptr_new.reshape(B))
